# Initial kernel scaffold; baseline (speedup 1.0000x reference)
import jax
import jax.numpy as jnp
from jax.experimental import pallas as pl


def kernel(x, table):
    raise NotImplementedError("write your pallas kernel here")


# speedup vs baseline: 1.4827x; 1.4827x over previous
import functools

import jax
import jax.numpy as jnp
from jax import lax
from jax.experimental import pallas as pl
from jax.experimental.pallas import tpu as pltpu
from jax.experimental.pallas import tpu_sc as plsc

VOCAB = 1000000
EMBED_DIM = 32
BATCH = 4096
SEQ = 200

NC = 2
NS = 16
NW = NC * NS

N = BATCH * SEQ
PER_W = N // NW
CH = 128
N_CH = PER_W // CH
NBUF = 4
N_REV = N_CH // NBUF


def _body(x_hbm, table_hbm, out_hbm, idx_v, bufs, gsems, wsems, isem):
    wid = lax.axis_index("s") * NC + lax.axis_index("c")
    base = wid * PER_W

    pltpu.async_copy(x_hbm.at[wid], idx_v, isem).wait()

    def fire_gather(j, b):
        return pltpu.async_copy(table_hbm.at[idx_v.at[j]], bufs[b], gsems[b])

    def wait_gather(b):
        pltpu.make_async_copy(table_hbm.at[idx_v.at[0]], bufs[b], gsems[b]).wait()

    def fire_write(j, b):
        return pltpu.async_copy(bufs[b], out_hbm.at[pl.ds(base + j * CH, CH)],
                                wsems[b])

    def wait_write(b):
        pltpu.make_async_copy(bufs[b], out_hbm.at[pl.ds(base, CH)],
                              wsems[b]).wait()

    for b in range(NBUF):
        fire_gather(b, b)

    def rev(g, carry):
        for b in range(NBUF):
            j = g * NBUF + b
            wait_gather(b)
            fire_write(j, b)
        for b in range(NBUF):
            j = g * NBUF + b
            wait_write(b)

            @pl.when(g < N_REV - 1)
            def _():
                fire_gather(j + NBUF, b)

        return carry

    lax.fori_loop(0, N_REV, rev, 0)


@functools.partial(jax.jit, static_argnames=())
def kernel(x, table):
    x_r = x.reshape(NW, N_CH, CH)
    mesh = plsc.VectorSubcoreMesh(core_axis_name="c", subcore_axis_name="s")
    out = pl.kernel(
        _body,
        out_type=jax.ShapeDtypeStruct((N, EMBED_DIM), jnp.float32),
        mesh=mesh,
        compiler_params=pltpu.CompilerParams(use_tc_tiling_on_sc=False),
        scratch_types=[
            pltpu.VMEM((N_CH, CH), jnp.int32),
            [pltpu.VMEM((CH, EMBED_DIM), jnp.float32) for _ in range(NBUF)],
            [pltpu.SemaphoreType.DMA for _ in range(NBUF)],
            [pltpu.SemaphoreType.DMA for _ in range(NBUF)],
            pltpu.SemaphoreType.DMA,
        ],
    )(x_r, table)
    return out.reshape(BATCH, SEQ, EMBED_DIM)


# speedup vs baseline: 1.4980x; 1.0103x over previous
import functools

import jax
import jax.numpy as jnp
from jax import lax
from jax.experimental import pallas as pl
from jax.experimental.pallas import tpu as pltpu
from jax.experimental.pallas import tpu_sc as plsc

VOCAB = 1000000
EMBED_DIM = 32
BATCH = 4096
SEQ = 200

NC = 2
NS = 16
NW = NC * NS

N = BATCH * SEQ
PER_W = N // NW
CH = 128
N_CH = PER_W // CH
NBUF = 8
N_REV = N_CH // NBUF


def _body(x_hbm, table_hbm, out_hbm, idx_v, bufs, gsems, wsems, isem):
    wid = lax.axis_index("s") * NC + lax.axis_index("c")
    base = wid * PER_W

    pltpu.async_copy(x_hbm.at[wid], idx_v, isem).wait()

    def fire_gather(j, b):
        return pltpu.async_copy(table_hbm.at[idx_v.at[j]], bufs[b], gsems[b])

    def wait_gather(b):
        pltpu.make_async_copy(table_hbm.at[idx_v.at[0]], bufs[b], gsems[b]).wait()

    def fire_write(j, b):
        return pltpu.async_copy(bufs[b], out_hbm.at[pl.ds(base + j * CH, CH)],
                                wsems[b])

    def wait_write(b):
        pltpu.make_async_copy(bufs[b], out_hbm.at[pl.ds(base, CH)],
                              wsems[b]).wait()

    for b in range(NBUF):
        fire_gather(b, b)

    def rev(g, carry):
        for b in range(NBUF):
            j = g * NBUF + b
            wait_gather(b)
            fire_write(j, b)
        for b in range(NBUF):
            j = g * NBUF + b
            wait_write(b)

            @pl.when(g < N_REV - 1)
            def _():
                fire_gather(j + NBUF, b)

        return carry

    lax.fori_loop(0, N_REV, rev, 0)


@functools.partial(jax.jit, static_argnames=())
def kernel(x, table):
    x_r = x.reshape(NW, N_CH, CH)
    mesh = plsc.VectorSubcoreMesh(core_axis_name="c", subcore_axis_name="s")
    out = pl.kernel(
        _body,
        out_type=jax.ShapeDtypeStruct((N, EMBED_DIM), jnp.float32),
        mesh=mesh,
        compiler_params=pltpu.CompilerParams(use_tc_tiling_on_sc=False),
        scratch_types=[
            pltpu.VMEM((N_CH, CH), jnp.int32),
            [pltpu.VMEM((CH, EMBED_DIM), jnp.float32) for _ in range(NBUF)],
            [pltpu.SemaphoreType.DMA for _ in range(NBUF)],
            [pltpu.SemaphoreType.DMA for _ in range(NBUF)],
            pltpu.SemaphoreType.DMA,
        ],
    )(x_r, table)
    return out.reshape(BATCH, SEQ, EMBED_DIM)
